# baseline (speedup 1.0000x reference)
import functools

import jax
import jax.numpy as jnp
from jax import lax
from jax.experimental import pallas as pl
from jax.experimental.pallas import tpu as pltpu
from jax.experimental.pallas import tpu_sc as plsc

B = 4
S = 2048
D = 768

_info = plsc.get_sparse_core_info()
_NC = _info.num_cores
_NS = _info.num_subcores
_L = _info.num_lanes
_NW = _NC * _NS
_S_PER_W = S // _NW
_C = 16
_CPB = _S_PER_W // _C
_NCHUNK = B * _CPB
_NBUF = 6
_DEPTH = 4
_VECS = D // _L

_mesh = plsc.VectorSubcoreMesh(core_axis_name="c", subcore_axis_name="s")


@functools.partial(
    pl.kernel,
    mesh=_mesh,
    out_type=jax.ShapeDtypeStruct((B * S, D), jnp.float32),
    scratch_types=[
        pltpu.VMEM((B * _S_PER_W,), jnp.int32),
        pltpu.VMEM((_S_PER_W, D), jnp.float32),
        pltpu.VMEM((_NBUF * _C, D), jnp.float32),
        pltpu.SemaphoreType.DMA((_NBUF,)),
        pltpu.SemaphoreType.DMA((_NBUF,)),
        pltpu.SemaphoreType.DMA,
    ],
)
def _embed(ids_hbm, word_hbm, pos_hbm, out_hbm,
           idx_v, pos_v, ring, gsem, wsem, psem):
    wid = lax.axis_index("s") * _NC + lax.axis_index("c")
    s_base = wid * _S_PER_W

    pos_load = pltpu.async_copy(pos_hbm.at[pl.ds(s_base, _S_PER_W)],
                                pos_v, psem)
    for b in range(B):
        pltpu.sync_copy(ids_hbm.at[b, pl.ds(s_base, _S_PER_W)],
                        idx_v.at[pl.ds(b * _S_PER_W, _S_PER_W)])

    def gather_desc(j, k):
        return pltpu.make_async_copy(
            word_hbm.at[idx_v.at[pl.ds(j * _C, _C)]],
            ring.at[pl.ds(k * _C, _C)],
            gsem.at[k])

    def write_desc(j, k):
        r0 = (j >> 2) * S + s_base + (j & 3) * _C
        return pltpu.make_async_copy(
            ring.at[pl.ds(k * _C, _C)],
            out_hbm.at[pl.ds(r0, _C)],
            wsem.at[k])

    for j in range(_DEPTH):
        gather_desc(j, j).start()

    pos_load.wait()

    def step(j, carry, fire_next=False):
        k = lax.rem(j, _NBUF)
        gather_desc(j, k).wait()
        h = (j & 3) * _C

        def _row(r, carry):
            for c in range(_VECS):
                sl = pl.ds(c * _L, _L)
                plsc.addupdate(ring.at[k * _C + r, sl], pos_v[h + r, sl])
            return carry

        lax.fori_loop(0, _C, _row, 0)
        write_desc(j, k).start()
        if fire_next:
            kn = lax.rem(j + _DEPTH, _NBUF)

            @pl.when(j >= _NBUF - _DEPTH)
            def _():
                write_desc(j - (_NBUF - _DEPTH), kn).wait()

            gather_desc(j + _DEPTH, kn).start()
        return 0

    lax.fori_loop(0, _NCHUNK - _DEPTH, functools.partial(step, fire_next=True),
                  0, unroll=False)
    lax.fori_loop(_NCHUNK - _DEPTH, _NCHUNK,
                  functools.partial(step, fire_next=False), 0, unroll=False)

    def drain(i, carry):
        k = lax.rem(i, _NBUF)
        pltpu.make_async_copy(ring.at[pl.ds(k * _C, _C)],
                              out_hbm.at[pl.ds(s_base, _C)],
                              wsem.at[k]).wait()
        return carry

    lax.fori_loop(_NCHUNK - _NBUF, _NCHUNK, drain, 0, unroll=False)


def kernel(ids, word_table, pos_table):
    out = _embed(ids.astype(jnp.int32), word_table, pos_table)
    return out.reshape(B, S, D)
